# baseline (speedup 1.0000x reference)
import functools

import jax
import jax.numpy as jnp
import numpy as np
from jax import lax
from jax.experimental import pallas as pl
from jax.experimental.pallas import tpu as pltpu
from jax.experimental.pallas import tpu_sc as plsc

N_ANCHORS = 20000
LANES = 128
ROWS = 160
NPAD = ROWS * LANES
PRE_NMS = 6000
N_OUT = 1000
IOU_THR = 0.7
NEG = np.float32(-1e38)

SC_TILES = 16
SC_PART = NPAD // SC_TILES
SC_CHUNKS = SC_PART // 128
COMP = 6144
CROWS = COMP // LANES
TRASH = 2 * COMP
COMP_ROWS = TRASH + 8
ROWW = 16


def _threshold_kernel(scores_ref, out_ref, outi_ref):
    scores = scores_ref[0]
    bits = lax.bitcast_convert_type(scores, jnp.int32)

    def count_ge(v):
        return jnp.sum((bits >= v).astype(jnp.int32))

    def bs_body(_, state):
        lo, hi = state
        mid = lo + (hi - lo) // 2
        ge = count_ge(mid) >= PRE_NMS
        return (jnp.where(ge, mid, lo), jnp.where(ge, hi, mid))

    lo, _ = lax.fori_loop(
        0, 31, bs_body, (jnp.int32(0), jnp.int32(np.int32(0x7F800000))))
    vstar = lo
    count_gt = jnp.sum((bits > vstar).astype(jnp.int32))
    k_ties = PRE_NMS - count_gt

    row_iota = lax.broadcasted_iota(jnp.int32, (ROWS, LANES), 0)
    col_iota = lax.broadcasted_iota(jnp.int32, (ROWS, LANES), 1)
    idx2d = row_iota * LANES + col_iota
    is_tie = bits == vstar

    def count_tie_lt(i):
        return jnp.sum((is_tie & (idx2d < i)).astype(jnp.int32))

    def bs2_body(_, state):
        lo2, hi2 = state
        mid = lo2 + (hi2 - lo2) // 2
        ge = count_tie_lt(mid) >= k_ties
        return (jnp.where(ge, lo2, mid), jnp.where(ge, mid, hi2))

    _, hi2 = lax.fori_loop(0, 15, bs2_body, (jnp.int32(0), jnp.int32(NPAD)))
    istar = hi2

    lane = lax.broadcasted_iota(jnp.int32, (1, LANES), 1)
    vstar_f = lax.bitcast_convert_type(vstar, jnp.float32)
    out_ref[0] = jnp.where(lane == 0, vstar_f, np.float32(0.0))
    outi_ref[0] = jnp.where(lane == 0, istar, 0)


def _compact_kernel(scores_hbm, geom_hbm, thrf_hbm, thri_hbm, comp_hbm,
                    sco_v, geo_v, thrf_v, thri_v, rows_v, idx_v, cnt_v,
                    counts_sh, allcnt_v, sem):
    b = lax.axis_index("c")
    part = lax.axis_index("s")
    base = part * SC_PART

    pltpu.sync_copy(scores_hbm.at[b, pl.ds(base, SC_PART)], sco_v)
    for ch in range(8):
        pltpu.sync_copy(geom_hbm.at[b, ch, pl.ds(base, SC_PART)],
                        geo_v.at[ch])
    pltpu.sync_copy(thrf_hbm.at[b], thrf_v)
    pltpu.sync_copy(thri_hbm.at[b], thri_v)
    vstar = thrf_v[pl.ds(0, 16)][0]
    istar = thri_v[pl.ds(0, 16)][0]
    lane = lax.broadcasted_iota(jnp.int32, (16,), 0)

    def active_mask(off):
        s16 = sco_v[pl.ds(off, 16)]
        gidx = base + off + lane
        return s16, (s16 > vstar) | ((s16 == vstar) & (gidx < istar))

    acc = jnp.zeros((16,), jnp.float32)
    for i in range(SC_PART // 16):
        _, m = active_mask(i * 16)
        acc = acc + jnp.where(m, jnp.float32(1), jnp.float32(0))
    cnt_v[...] = jnp.full((16,), jnp.sum(acc).astype(jnp.int32), jnp.int32)
    pltpu.sync_copy(cnt_v, counts_sh.at[part])
    plsc.subcore_barrier()
    pltpu.sync_copy(counts_sh, allcnt_v)

    goff = b * COMP
    for p in range(SC_TILES):
        cvec = allcnt_v[p, pl.ds(0, 16)]
        goff = goff + jnp.where(p < part, cvec[0], 0)

    one = np.float32(1.0)
    zero = np.float32(0.0)
    half = np.float32(0.5)
    handles = []
    cc = jnp.int32(0)
    for j in range(SC_CHUNKS):
        for kk in range(8):
            off = j * 128 + kk * 16
            s16, m = active_mask(off)
            sl = pl.ds(off, 16)
            ay1 = geo_v[0, sl]
            ax1 = geo_v[1, sl]
            ay2 = geo_v[2, sl]
            ax2 = geo_v[3, sl]
            dy = geo_v[4, sl] * np.float32(0.1)
            dx = geo_v[5, sl] * np.float32(0.1)
            dh = geo_v[6, sl] * np.float32(0.2)
            dw = geo_v[7, sl] * np.float32(0.2)
            height = ay2 - ay1
            width = ax2 - ax1
            center_y = ay1 + half * height
            center_x = ax1 + half * width
            center_y = center_y + dy * height
            center_x = center_x + dx * width
            height = height * jnp.exp(dh)
            width = width * jnp.exp(dw)
            y1 = center_y - half * height
            x1 = center_x - half * width
            y2 = y1 + height
            x2 = x1 + width
            y1 = jnp.maximum(jnp.minimum(y1, one), zero)
            x1 = jnp.maximum(jnp.minimum(x1, one), zero)
            y2 = jnp.maximum(jnp.minimum(y2, one), zero)
            x2 = jnp.maximum(jnp.minimum(x2, one), zero)
            area = (y2 - y1) * (x2 - x1)

            mf = jnp.where(m, jnp.float32(1), jnp.float32(0))
            excl = (plsc.cumsum(mf) - mf).astype(jnp.int32)
            rowidx = off + lane
            vals = (s16, y1, x1, y2, x2, area)
            for ch, v in enumerate(vals):
                plsc.store_scatter(
                    rows_v, [rowidx, jnp.full((16,), ch, jnp.int32)], v)
            idx_v[j, pl.ds(kk * 16, 16)] = jnp.where(
                m, goff + cc + excl, jnp.int32(TRASH))
            cc = cc + jnp.sum(mf).astype(jnp.int32)
        handles.append(pltpu.async_copy(
            rows_v.at[pl.ds(j * 128, 128)], comp_hbm.at[idx_v.at[j]], sem))
    for h in handles:
        h.wait()


_compact = functools.partial(
    pl.kernel,
    out_type=jax.ShapeDtypeStruct((COMP_ROWS, ROWW), jnp.float32),
    mesh=plsc.VectorSubcoreMesh(core_axis_name="c", subcore_axis_name="s"),
    compiler_params=pltpu.CompilerParams(
        needs_layout_passes=False, use_tc_tiling_on_sc=False),
    scratch_types=[
        pltpu.VMEM((SC_PART,), jnp.float32),
        pltpu.VMEM((8, SC_PART), jnp.float32),
        pltpu.VMEM((LANES,), jnp.float32),
        pltpu.VMEM((LANES,), jnp.int32),
        pltpu.VMEM((SC_PART, ROWW), jnp.float32),
        pltpu.VMEM((SC_CHUNKS, 128), jnp.int32),
        pltpu.VMEM((16,), jnp.int32),
        pltpu.VMEM_SHARED((SC_TILES, 16), jnp.int32),
        pltpu.VMEM((SC_TILES, 16), jnp.int32),
        pltpu.SemaphoreType.DMA,
    ],
)(_compact_kernel)


def _nms_kernel(comp_ref, raw_ref, out_ref):
    batch = comp_ref.shape[0]
    row_iota = lax.broadcasted_iota(jnp.int32, (CROWS, LANES), 0)
    col_iota = lax.broadcasted_iota(jnp.int32, (CROWS, LANES), 1)
    idx2d = row_iota * LANES + col_iota
    idx2df = idx2d.astype(jnp.float32)
    zero = np.float32(0.0)
    out_iota = (lax.broadcasted_iota(jnp.int32, (8, LANES), 0) * LANES
                + lax.broadcasted_iota(jnp.int32, (8, LANES), 1))
    thr = np.float32(IOU_THR)
    big = jnp.int32(2 ** 30)
    fz = jnp.float32(0.0)

    masked0 = tuple(
        jnp.where(idx2d < PRE_NMS, comp_ref[b, 0], NEG) for b in range(batch))
    outs0 = tuple(
        jnp.zeros((8, LANES), jnp.float32) for _ in range(4 * batch))

    def nms_body(i, carry):
        maskeds = carry[:batch]
        outs = list(carry[batch:])
        selo = out_iota == i
        sub_iota = lax.broadcasted_iota(jnp.int32, (8, LANES), 0)

        def halves_combine(vals, fold, fill):
            h = []
            for b in range(batch):
                v8 = fold(vals[b].reshape(6, 8, LANES), 0)
                h.append(fold(jnp.stack(
                    [v8, pltpu.roll(v8, 4, 0)], 0), 0))
            comb = jnp.where(sub_iota < 4, h[0], pltpu.roll(h[1], 4, 0))
            return comb

        def splat_half(red, b, fill, fold2):
            rb = jnp.broadcast_to(red, (8, LANES))
            half = (sub_iota < 4) if b == 0 else (sub_iota >= 4)
            v = jnp.where(half, rb, fill)
            v = fold2(v, pltpu.roll(v, 4, 0))
            v = fold2(v, pltpu.roll(v, 2, 0))
            v = fold2(v, pltpu.roll(v, 1, 0))
            return v

        combm = halves_combine(maskeds, jnp.max, NEG)
        mred = jnp.max(combm, axis=1, keepdims=True)
        m8s = [splat_half(mred, 0, NEG, jnp.maximum),
               splat_half(mred, 1, NEG, jnp.maximum)]
        valids8 = [m8s[b] >= zero for b in range(batch)]
        valids = [jnp.broadcast_to(
            valids8[b].reshape(1, 8, LANES),
            (6, 8, LANES)).reshape(CROWS, LANES) for b in range(batch)]
        mbs = [jnp.broadcast_to(
            m8s[b].reshape(1, 8, LANES),
            (6, 8, LANES)).reshape(CROWS, LANES) for b in range(batch)]
        sels = [maskeds[b] == mbs[b] for b in range(batch)]

        bigf = jnp.float32(1e9)
        cands = [jnp.where(sels[b], idx2df, bigf) for b in range(batch)]
        combc = halves_combine(cands, jnp.min, bigf)
        cred = jnp.min(combc, axis=1, keepdims=True)
        cfin = [splat_half(cred, 0, bigf, jnp.minimum),
                splat_half(cred, 1, bigf, jnp.minimum)]
        js = [jnp.minimum(cfin[0][0, 0], jnp.float32(COMP - 1))
              .astype(jnp.int32),
              jnp.minimum(cfin[1][0, 0], jnp.float32(COMP - 1))
              .astype(jnp.int32)]
        rows = [raw_ref[b, pl.ds(js[b], 1), :] for b in range(batch)]
        new_masked = []
        for b in range(batch):
            row = rows[b]
            by1 = row[0:1, 1:2]
            bx1 = row[0:1, 2:3]
            by2 = row[0:1, 3:4]
            bx2 = row[0:1, 4:5]
            barea = row[0:1, 5:6]
            selj = idx2d == js[b]
            cy1 = comp_ref[b, 1]
            cx1 = comp_ref[b, 2]
            cy2 = comp_ref[b, 3]
            cx2 = comp_ref[b, 4]
            car = comp_ref[b, 5]
            yy1 = jnp.maximum(by1, cy1)
            xx1 = jnp.maximum(bx1, cx1)
            yy2 = jnp.minimum(by2, cy2)
            xx2 = jnp.minimum(bx2, cx2)
            inter = (jnp.maximum(yy2 - yy1, zero)
                     * jnp.maximum(xx2 - xx1, zero))
            union = barea + car - inter
            iou = jnp.where(union > zero, inter / union, zero)
            suppress = ((iou > thr) | selj) & valids[b]
            new_masked.append(jnp.where(suppress, NEG, maskeds[b]))
            wsel = selo & valids8[b]
            outs[4 * b + 0] = jnp.where(wsel, by1, outs[4 * b + 0])
            outs[4 * b + 1] = jnp.where(wsel, bx1, outs[4 * b + 1])
            outs[4 * b + 2] = jnp.where(wsel, by2, outs[4 * b + 2])
            outs[4 * b + 3] = jnp.where(wsel, bx2, outs[4 * b + 3])
        return tuple(new_masked) + tuple(outs)

    fin = lax.fori_loop(0, N_OUT, nms_body, masked0 + outs0)
    for b in range(batch):
        for ch in range(4):
            out_ref[b, ch] = fin[batch + 4 * b + ch]


@jax.jit
def kernel(rpn_probs, rpn_bbox, anchors):
    batch = rpn_probs.shape[0]
    scores = rpn_probs[:, :, 1]
    scores = jnp.pad(scores, ((0, 0), (0, NPAD - N_ANCHORS)),
                     constant_values=-1.0)
    geom = jnp.concatenate(
        [anchors.transpose(0, 2, 1), rpn_bbox.transpose(0, 2, 1)], axis=1)
    geom = jnp.pad(geom, ((0, 0), (0, 0), (0, NPAD - N_ANCHORS)))

    thrf, thri = pl.pallas_call(
        _threshold_kernel,
        grid=(batch,),
        in_specs=[pl.BlockSpec((1, ROWS, LANES), lambda b: (b, 0, 0))],
        out_specs=[pl.BlockSpec((1, 1, LANES), lambda b: (b, 0, 0)),
                   pl.BlockSpec((1, 1, LANES), lambda b: (b, 0, 0))],
        out_shape=[jax.ShapeDtypeStruct((batch, 1, LANES), jnp.float32),
                   jax.ShapeDtypeStruct((batch, 1, LANES), jnp.int32)],
    )(scores.reshape(batch, ROWS, LANES))

    comp = _compact(scores, geom, thrf.reshape(batch, LANES),
                    thri.reshape(batch, LANES))

    raw = comp[:TRASH].reshape(batch, COMP, ROWW)
    compt = raw.transpose(0, 2, 1)[:, :6, :].reshape(batch, 6, CROWS, LANES)

    out = pl.pallas_call(
        _nms_kernel,
        out_shape=jax.ShapeDtypeStruct((batch, 4, 8, LANES), jnp.float32),
    )(compt, raw)

    out = out.reshape(batch, 4, 8 * LANES)[:, :, :N_OUT]
    return out.transpose(0, 2, 1)


# speedup vs baseline: 1.2199x; 1.2199x over previous
import functools

import jax
import jax.numpy as jnp
import numpy as np
from jax import lax
from jax.experimental import pallas as pl
from jax.experimental.pallas import tpu as pltpu
from jax.experimental.pallas import tpu_sc as plsc

N_ANCHORS = 20000
LANES = 128
ROWS = 160
NPAD = ROWS * LANES
PRE_NMS = 6000
N_OUT = 1000
IOU_THR = 0.7
NEG = np.float32(-1e38)

SC_TILES = 16
SC_PART = NPAD // SC_TILES
SC_CHUNKS = SC_PART // 128
COMP = 6144
CROWS = COMP // LANES
TRASH = 2 * COMP
COMP_ROWS = TRASH + 40
ROWW = 16


def _threshold_kernel(scores_ref, out_ref, outi_ref):
    scores = scores_ref[0]
    bits = lax.bitcast_convert_type(scores, jnp.int32)

    def count_ge(v):
        return jnp.sum((bits >= v).astype(jnp.int32))

    def bs_body(_, state):
        lo, hi = state
        mid = lo + (hi - lo) // 2
        ge = count_ge(mid) >= PRE_NMS
        return (jnp.where(ge, mid, lo), jnp.where(ge, hi, mid))

    lo, _ = lax.fori_loop(
        0, 31, bs_body, (jnp.int32(0), jnp.int32(np.int32(0x7F800000))))
    vstar = lo
    count_gt = jnp.sum((bits > vstar).astype(jnp.int32))
    k_ties = PRE_NMS - count_gt

    row_iota = lax.broadcasted_iota(jnp.int32, (ROWS, LANES), 0)
    col_iota = lax.broadcasted_iota(jnp.int32, (ROWS, LANES), 1)
    idx2d = row_iota * LANES + col_iota
    is_tie = bits == vstar

    def count_tie_lt(i):
        return jnp.sum((is_tie & (idx2d < i)).astype(jnp.int32))

    def bs2_body(_, state):
        lo2, hi2 = state
        mid = lo2 + (hi2 - lo2) // 2
        ge = count_tie_lt(mid) >= k_ties
        return (jnp.where(ge, lo2, mid), jnp.where(ge, mid, hi2))

    _, hi2 = lax.fori_loop(0, 15, bs2_body, (jnp.int32(0), jnp.int32(NPAD)))
    istar = hi2

    lane = lax.broadcasted_iota(jnp.int32, (1, LANES), 1)
    vstar_f = lax.bitcast_convert_type(vstar, jnp.float32)
    out_ref[0] = jnp.where(lane == 0, vstar_f, np.float32(0.0))
    outi_ref[0] = jnp.where(lane == 0, istar, 0)


def _compact_kernel(scores_hbm, geom_hbm, thrf_hbm, thri_hbm, comp_hbm,
                    sco_v, geo_v, thrf_v, thri_v, rows_v, idx_v, cnt_v,
                    counts_sh, allcnt_v, sem, gsem):
    b = lax.axis_index("c")
    part = lax.axis_index("s")
    base = part * SC_PART

    in_handles = [
        pltpu.async_copy(scores_hbm.at[b, pl.ds(base, SC_PART)], sco_v, sem),
        pltpu.async_copy(thrf_hbm.at[b], thrf_v, sem),
        pltpu.async_copy(thri_hbm.at[b], thri_v, sem),
    ]
    geom_handles = [
        pltpu.async_copy(geom_hbm.at[b, ch, pl.ds(base, SC_PART)],
                         geo_v.at[ch], gsem)
        for ch in range(8)
    ]
    for h in in_handles:
        h.wait()
    vstar = thrf_v[pl.ds(0, 16)][0]
    istar = thri_v[pl.ds(0, 16)][0]
    lane = lax.broadcasted_iota(jnp.int32, (16,), 0)

    def active_mask(off):
        s16 = sco_v[pl.ds(off, 16)]
        gidx = base + off + lane
        return s16, (s16 > vstar) | ((s16 == vstar) & (gidx < istar))

    acc = jnp.zeros((16,), jnp.float32)
    for i in range(SC_PART // 16):
        _, m = active_mask(i * 16)
        acc = acc + jnp.where(m, jnp.float32(1), jnp.float32(0))
    cnt_v[...] = jnp.full((16,), jnp.sum(acc).astype(jnp.int32), jnp.int32)
    pltpu.sync_copy(cnt_v, counts_sh.at[part])
    plsc.subcore_barrier()
    pltpu.sync_copy(counts_sh, allcnt_v)

    goff = b * COMP
    for p in range(SC_TILES):
        cvec = allcnt_v[p, pl.ds(0, 16)]
        goff = goff + jnp.where(p < part, cvec[0], 0)

    for h in geom_handles:
        h.wait()
    trash_row = TRASH + part + SC_TILES * b
    one = np.float32(1.0)
    zero = np.float32(0.0)
    half = np.float32(0.5)
    handles = []
    cc = jnp.int32(0)
    for j in range(SC_CHUNKS):
        for kk in range(8):
            off = j * 128 + kk * 16
            s16, m = active_mask(off)
            sl = pl.ds(off, 16)
            ay1 = geo_v[0, sl]
            ax1 = geo_v[1, sl]
            ay2 = geo_v[2, sl]
            ax2 = geo_v[3, sl]
            dy = geo_v[4, sl] * np.float32(0.1)
            dx = geo_v[5, sl] * np.float32(0.1)
            dh = geo_v[6, sl] * np.float32(0.2)
            dw = geo_v[7, sl] * np.float32(0.2)
            height = ay2 - ay1
            width = ax2 - ax1
            center_y = ay1 + half * height
            center_x = ax1 + half * width
            center_y = center_y + dy * height
            center_x = center_x + dx * width
            height = height * jnp.exp(dh)
            width = width * jnp.exp(dw)
            y1 = center_y - half * height
            x1 = center_x - half * width
            y2 = y1 + height
            x2 = x1 + width
            y1 = jnp.maximum(jnp.minimum(y1, one), zero)
            x1 = jnp.maximum(jnp.minimum(x1, one), zero)
            y2 = jnp.maximum(jnp.minimum(y2, one), zero)
            x2 = jnp.maximum(jnp.minimum(x2, one), zero)
            area = (y2 - y1) * (x2 - x1)

            mf = jnp.where(m, jnp.float32(1), jnp.float32(0))
            excl = (plsc.cumsum(mf) - mf).astype(jnp.int32)
            rowidx = off + lane
            vals = (s16, y1, x1, y2, x2, area)
            for ch, v in enumerate(vals):
                plsc.store_scatter(
                    rows_v, [rowidx, jnp.full((16,), ch, jnp.int32)], v)
            idx_v[j, pl.ds(kk * 16, 16)] = jnp.where(
                m, goff + cc + excl, trash_row)
            cc = cc + jnp.sum(mf).astype(jnp.int32)
        handles.append(pltpu.async_copy(
            rows_v.at[pl.ds(j * 128, 128)], comp_hbm.at[idx_v.at[j]], sem))
    for h in handles:
        h.wait()


_compact = functools.partial(
    pl.kernel,
    out_type=jax.ShapeDtypeStruct((COMP_ROWS, ROWW), jnp.float32),
    mesh=plsc.VectorSubcoreMesh(core_axis_name="c", subcore_axis_name="s"),
    compiler_params=pltpu.CompilerParams(
        needs_layout_passes=False, use_tc_tiling_on_sc=False),
    scratch_types=[
        pltpu.VMEM((SC_PART,), jnp.float32),
        pltpu.VMEM((8, SC_PART), jnp.float32),
        pltpu.VMEM((LANES,), jnp.float32),
        pltpu.VMEM((LANES,), jnp.int32),
        pltpu.VMEM((SC_PART, ROWW), jnp.float32),
        pltpu.VMEM((SC_CHUNKS, 128), jnp.int32),
        pltpu.VMEM((16,), jnp.int32),
        pltpu.VMEM_SHARED((SC_TILES, 16), jnp.int32),
        pltpu.VMEM((SC_TILES, 16), jnp.int32),
        pltpu.SemaphoreType.DMA,
        pltpu.SemaphoreType.DMA,
    ],
)(_compact_kernel)


def _nms_kernel(comp_ref, raw_ref, out_ref):
    batch = comp_ref.shape[0]
    row_iota = lax.broadcasted_iota(jnp.int32, (CROWS, LANES), 0)
    col_iota = lax.broadcasted_iota(jnp.int32, (CROWS, LANES), 1)
    idx2d = row_iota * LANES + col_iota
    idx2df = idx2d.astype(jnp.float32)
    zero = np.float32(0.0)
    out_iota = (lax.broadcasted_iota(jnp.int32, (8, LANES), 0) * LANES
                + lax.broadcasted_iota(jnp.int32, (8, LANES), 1))
    thr = np.float32(IOU_THR)
    big = jnp.int32(2 ** 30)
    fz = jnp.float32(0.0)

    masked0 = tuple(
        jnp.where(idx2d < PRE_NMS, comp_ref[b, 0], NEG) for b in range(batch))
    outs0 = tuple(
        jnp.zeros((8, LANES), jnp.float32) for _ in range(4 * batch))

    def nms_body(i, carry):
        maskeds = carry[:batch]
        outs = list(carry[batch:])
        selo = out_iota == i
        sub_iota = lax.broadcasted_iota(jnp.int32, (8, LANES), 0)

        def halves_combine(vals, fold, fill):
            h = []
            for b in range(batch):
                v8 = fold(vals[b].reshape(6, 8, LANES), 0)
                h.append(fold(jnp.stack(
                    [v8, pltpu.roll(v8, 4, 0)], 0), 0))
            comb = jnp.where(sub_iota < 4, h[0], pltpu.roll(h[1], 4, 0))
            return comb

        def splat_half(red, b, fill, fold2):
            rb = jnp.broadcast_to(red, (8, LANES))
            half = (sub_iota < 4) if b == 0 else (sub_iota >= 4)
            v = jnp.where(half, rb, fill)
            v = fold2(v, pltpu.roll(v, 4, 0))
            v = fold2(v, pltpu.roll(v, 2, 0))
            v = fold2(v, pltpu.roll(v, 1, 0))
            return v

        combm = halves_combine(maskeds, jnp.max, NEG)
        mred = jnp.max(combm, axis=1, keepdims=True)
        m8s = [splat_half(mred, 0, NEG, jnp.maximum),
               splat_half(mred, 1, NEG, jnp.maximum)]
        valids8 = [m8s[b] >= zero for b in range(batch)]
        valids = [jnp.broadcast_to(
            valids8[b].reshape(1, 8, LANES),
            (6, 8, LANES)).reshape(CROWS, LANES) for b in range(batch)]
        mbs = [jnp.broadcast_to(
            m8s[b].reshape(1, 8, LANES),
            (6, 8, LANES)).reshape(CROWS, LANES) for b in range(batch)]
        sels = [maskeds[b] == mbs[b] for b in range(batch)]

        bigf = jnp.float32(1e9)
        cands = [jnp.where(sels[b], idx2df, bigf) for b in range(batch)]
        combc = halves_combine(cands, jnp.min, bigf)
        cred = jnp.min(combc, axis=1, keepdims=True)
        cfin = [splat_half(cred, 0, bigf, jnp.minimum),
                splat_half(cred, 1, bigf, jnp.minimum)]
        js = [jnp.minimum(cfin[0][0, 0], jnp.float32(COMP - 1))
              .astype(jnp.int32),
              jnp.minimum(cfin[1][0, 0], jnp.float32(COMP - 1))
              .astype(jnp.int32)]
        rows = [raw_ref[b, pl.ds(js[b], 1), :] for b in range(batch)]
        new_masked = []
        for b in range(batch):
            row = rows[b]
            by1 = row[0:1, 1:2]
            bx1 = row[0:1, 2:3]
            by2 = row[0:1, 3:4]
            bx2 = row[0:1, 4:5]
            barea = row[0:1, 5:6]
            selj = idx2d == js[b]
            cy1 = comp_ref[b, 1]
            cx1 = comp_ref[b, 2]
            cy2 = comp_ref[b, 3]
            cx2 = comp_ref[b, 4]
            car = comp_ref[b, 5]
            yy1 = jnp.maximum(by1, cy1)
            xx1 = jnp.maximum(bx1, cx1)
            yy2 = jnp.minimum(by2, cy2)
            xx2 = jnp.minimum(bx2, cx2)
            inter = (jnp.maximum(yy2 - yy1, zero)
                     * jnp.maximum(xx2 - xx1, zero))
            union = barea + car - inter
            iou = jnp.where(union > zero, inter / union, zero)
            suppress = ((iou > thr) | selj) & valids[b]
            new_masked.append(jnp.where(suppress, NEG, maskeds[b]))
            wsel = selo & valids8[b]
            outs[4 * b + 0] = jnp.where(wsel, by1, outs[4 * b + 0])
            outs[4 * b + 1] = jnp.where(wsel, bx1, outs[4 * b + 1])
            outs[4 * b + 2] = jnp.where(wsel, by2, outs[4 * b + 2])
            outs[4 * b + 3] = jnp.where(wsel, bx2, outs[4 * b + 3])
        return tuple(new_masked) + tuple(outs)

    fin = lax.fori_loop(0, N_OUT, nms_body, masked0 + outs0)
    for b in range(batch):
        for ch in range(4):
            out_ref[b, ch] = fin[batch + 4 * b + ch]


@jax.jit
def kernel(rpn_probs, rpn_bbox, anchors):
    batch = rpn_probs.shape[0]
    scores = rpn_probs[:, :, 1]
    scores = jnp.pad(scores, ((0, 0), (0, NPAD - N_ANCHORS)),
                     constant_values=-1.0)
    geom = jnp.concatenate(
        [anchors.transpose(0, 2, 1), rpn_bbox.transpose(0, 2, 1)], axis=1)
    geom = jnp.pad(geom, ((0, 0), (0, 0), (0, NPAD - N_ANCHORS)))

    thrf, thri = pl.pallas_call(
        _threshold_kernel,
        grid=(batch,),
        in_specs=[pl.BlockSpec((1, ROWS, LANES), lambda b: (b, 0, 0))],
        out_specs=[pl.BlockSpec((1, 1, LANES), lambda b: (b, 0, 0)),
                   pl.BlockSpec((1, 1, LANES), lambda b: (b, 0, 0))],
        out_shape=[jax.ShapeDtypeStruct((batch, 1, LANES), jnp.float32),
                   jax.ShapeDtypeStruct((batch, 1, LANES), jnp.int32)],
    )(scores.reshape(batch, ROWS, LANES))

    comp = _compact(scores, geom, thrf.reshape(batch, LANES),
                    thri.reshape(batch, LANES))

    raw = comp[:TRASH].reshape(batch, COMP, ROWW)
    compt = raw.transpose(0, 2, 1)[:, :6, :].reshape(batch, 6, CROWS, LANES)

    out = pl.pallas_call(
        _nms_kernel,
        out_shape=jax.ShapeDtypeStruct((batch, 4, 8, LANES), jnp.float32),
    )(compt, raw)

    out = out.reshape(batch, 4, 8 * LANES)[:, :, :N_OUT]
    return out.transpose(0, 2, 1)
